# Initial kernel scaffold; baseline (speedup 1.0000x reference)
import jax
import jax.numpy as jnp
from jax.experimental import pallas as pl


def kernel(y_pred, component_ln_gammas, component_batch_batch):
    raise NotImplementedError("write your pallas kernel here")


# speedup vs baseline: 73.9888x; 73.9888x over previous
import functools

import jax
import jax.numpy as jnp
from jax import lax
from jax.experimental import pallas as pl
from jax.experimental.pallas import tpu as pltpu
from jax.experimental.pallas import tpu_sc as plsc

N_COMP = 6_400_000
N_MIX = 100_000
LANES = 16
N_CORES = 2
N_SUBCORES = 16
NW = N_CORES * N_SUBCORES
PER_W = N_COMP // NW
CHUNK = 20_000
N_CHUNKS = PER_W // CHUNK
VECS_PER_CHUNK = CHUNK // LANES


def _sc_partials(y, g):
    mesh = plsc.VectorSubcoreMesh(core_axis_name="c", subcore_axis_name="s")

    @functools.partial(
        pl.kernel,
        out_type=jax.ShapeDtypeStruct((NW, LANES), jnp.float32),
        mesh=mesh,
        scratch_types=[
            pltpu.VMEM((CHUNK,), jnp.float32),
            pltpu.VMEM((CHUNK,), jnp.float32),
            pltpu.VMEM((LANES,), jnp.float32),
        ],
    )
    def k(y_hbm, g_hbm, out_hbm, ybuf, gbuf, accbuf):
        wid = lax.axis_index("s") * N_CORES + lax.axis_index("c")
        base = wid * PER_W

        def chunk_body(ci, acc):
            off = base + ci * CHUNK
            pltpu.sync_copy(y_hbm.at[pl.ds(off, CHUNK)], ybuf)
            pltpu.sync_copy(g_hbm.at[pl.ds(off, CHUNK)], gbuf)

            def vec_body(i, a):
                d = ybuf[pl.ds(i * LANES, LANES)] - gbuf[pl.ds(i * LANES, LANES)]
                return a + d * d

            return lax.fori_loop(0, VECS_PER_CHUNK, vec_body, acc)

        acc = lax.fori_loop(0, N_CHUNKS, chunk_body, jnp.zeros((LANES,), jnp.float32))
        accbuf[...] = acc
        pltpu.sync_copy(accbuf, out_hbm.at[wid])

    return k(y, g)


def _reduce_partials(partials):
    def body(x_ref, o_ref):
        o_ref[0, 0] = jnp.sum(x_ref[...]) * (1.0 / N_MIX)

    return pl.pallas_call(
        body,
        out_shape=jax.ShapeDtypeStruct((1, 1), jnp.float32),
        out_specs=pl.BlockSpec(memory_space=pltpu.SMEM),
    )(partials)


def kernel(y_pred, component_ln_gammas, component_batch_batch):
    del component_batch_batch
    partials = _sc_partials(y_pred, component_ln_gammas)
    return _reduce_partials(partials)[0, 0]


# speedup vs baseline: 176.2106x; 2.3816x over previous
import functools

import jax
import jax.numpy as jnp
from jax import lax
from jax.experimental import pallas as pl
from jax.experimental.pallas import tpu as pltpu
from jax.experimental.pallas import tpu_sc as plsc

N_COMP = 6_400_000
N_MIX = 100_000
LANES = 16
N_CORES = 2
N_SUBCORES = 16
NW = N_CORES * N_SUBCORES
PER_W = N_COMP // NW
CHUNK = 20_000
N_CHUNKS = PER_W // CHUNK
VECS_PER_CHUNK = CHUNK // LANES
UNROLL = 10
N_ACC = 4


def _sc_partials(y, g):
    mesh = plsc.VectorSubcoreMesh(core_axis_name="c", subcore_axis_name="s")

    @functools.partial(
        pl.kernel,
        out_type=jax.ShapeDtypeStruct((NW, LANES), jnp.float32),
        mesh=mesh,
        scratch_types=[
            pltpu.VMEM((CHUNK,), jnp.float32),
            pltpu.VMEM((CHUNK,), jnp.float32),
            pltpu.VMEM((CHUNK,), jnp.float32),
            pltpu.VMEM((CHUNK,), jnp.float32),
            pltpu.VMEM((LANES,), jnp.float32),
            pltpu.SemaphoreType.DMA,
            pltpu.SemaphoreType.DMA,
            pltpu.SemaphoreType.DMA,
            pltpu.SemaphoreType.DMA,
        ],
    )
    def k(y_hbm, g_hbm, out_hbm, ybuf0, ybuf1, gbuf0, gbuf1, accbuf, sy0, sy1, sg0, sg1):
        wid = lax.axis_index("s") * N_CORES + lax.axis_index("c")
        base = wid * PER_W
        ybufs = (ybuf0, ybuf1)
        gbufs = (gbuf0, gbuf1)
        sy = (sy0, sy1)
        sg = (sg0, sg1)

        def start(c):
            slot = c % 2
            off = base + c * CHUNK
            pltpu.async_copy(y_hbm.at[pl.ds(off, CHUNK)], ybufs[slot], sy[slot])
            pltpu.async_copy(g_hbm.at[pl.ds(off, CHUNK)], gbufs[slot], sg[slot])

        def wait(c):
            slot = c % 2
            off = base + c * CHUNK
            pltpu.make_async_copy(y_hbm.at[pl.ds(off, CHUNK)], ybufs[slot], sy[slot]).wait()
            pltpu.make_async_copy(g_hbm.at[pl.ds(off, CHUNK)], gbufs[slot], sg[slot]).wait()

        start(0)
        start(1)
        accs = [jnp.zeros((LANES,), jnp.float32) for _ in range(N_ACC)]
        for c in range(N_CHUNKS):
            slot = c % 2
            wait(c)
            if c + 2 < N_CHUNKS:
                start(c + 2)
            yb = ybufs[slot]
            gb = gbufs[slot]

            def vec_body(i, accs, yb=yb, gb=gb):
                accs = list(accs)
                for u in range(UNROLL):
                    o = i * (UNROLL * LANES) + u * LANES
                    d = yb[pl.ds(o, LANES)] - gb[pl.ds(o, LANES)]
                    accs[u % N_ACC] = accs[u % N_ACC] + d * d
                return tuple(accs)

            accs = lax.fori_loop(0, VECS_PER_CHUNK // UNROLL, vec_body, tuple(accs))
            accs = list(accs)
        accbuf[...] = (accs[0] + accs[1]) + (accs[2] + accs[3])
        pltpu.sync_copy(accbuf, out_hbm.at[wid])

    return k(y, g)


def _reduce_partials(partials):
    def body(x_ref, o_ref):
        o_ref[0, 0] = jnp.sum(x_ref[...]) * (1.0 / N_MIX)

    return pl.pallas_call(
        body,
        out_shape=jax.ShapeDtypeStruct((1, 1), jnp.float32),
        out_specs=pl.BlockSpec(memory_space=pltpu.SMEM),
    )(partials)


def kernel(y_pred, component_ln_gammas, component_batch_batch):
    del component_batch_batch
    partials = _sc_partials(y_pred, component_ln_gammas)
    return _reduce_partials(partials)[0, 0]
